# baseline (speedup 1.0000x reference)
import functools

import jax
import jax.numpy as jnp
from jax import lax
from jax.experimental import pallas as pl
from jax.experimental.pallas import tpu as pltpu
from jax.experimental.pallas import tpu_sc as plsc

ROWS = 128
N = 32768
TOPK = 10
NC = 2
NS = 16
L = 16
NW = NC * NS
RPW = ROWS // NW
SB = 256
NSB = N // SB
BPS = SB // L
NBLK = N // L

_NEG = float("-inf")


def _pass1(ts, iota16):
  sv = []
  for g in range(8):
    def sb_body(s_, ssm_g, g=g):
      bases = [(g * 16 + 4 * s_ + u) * SB for u in range(4)]
      ms = [ts[pl.ds(b, L)] for b in bases]
      for j in range(1, SB // L):
        for u in range(4):
          ms[u] = jnp.maximum(ms[u], ts[pl.ds(bases[u] + j * L, L)])
      for u in range(4):
        ssm_g = jnp.where(iota16 == 4 * s_ + u, jnp.max(ms[u]), ssm_g)
      return ssm_g
    sv.append(lax.fori_loop(0, 4, sb_body, jnp.full((L,), _NEG, jnp.float32)))
  return sv


def _top10_cells(vecs, ids, nvec, sentinel, iota16):
  R = len(vecs)
  win = [jnp.zeros((L,), jnp.int32) for _ in range(R)]
  for k in range(TOPK):
    gv = [None] * R
    for r in range(R):
      m = vecs[r][0]
      for t in range(1, nvec):
        m = jnp.maximum(m, vecs[r][t])
      gv[r] = jnp.max(m)
    st = [None] * R
    for r in range(R):
      cidx = jnp.full((L,), sentinel, jnp.int32)
      for t in range(nvec):
        cidx = jnp.minimum(
            cidx, jnp.where(vecs[r][t] == gv[r], ids[r][t], sentinel))
      st[r] = jnp.min(cidx)
    for r in range(R):
      win[r] = jnp.where(iota16 == k, st[r], win[r])
      for t in range(nvec):
        vecs[r][t] = jnp.where(ids[r][t] == st[r], _NEG, vecs[r][t])
  return win


def _rows_topk(tss, svs, iota16):
  R = len(tss)

  sb_ids = [[iota16 + g * 16 for g in range(8)] for _ in range(R)]
  winsb = _top10_cells([list(sv) for sv in svs], sb_ids, 8, NSB, iota16)

  s_sc = [[jnp.max(jnp.where(iota16 == k, winsb[r], 0)) for k in range(TOPK)]
          for r in range(R)]

  bmv = [[] for _ in range(R)]
  bidv = [[] for _ in range(R)]
  for k in range(TOPK):
    for r in range(R):
      gidx = s_sc[r][k] * SB + iota16 * L
      bm = plsc.load_gather(tss[r], [gidx])
      for j in range(1, L):
        bm = jnp.maximum(bm, plsc.load_gather(tss[r], [gidx + j]))
      bmv[r].append(bm)
      bidv[r].append(s_sc[r][k] * BPS + iota16)

  winblk = _top10_cells(bmv, bidv, TOPK, NBLK, iota16)
  b_sc = [[jnp.max(jnp.where(iota16 == k, winblk[r], 0)) for k in range(TOPK)]
          for r in range(R)]

  cv = [[] for _ in range(R)]
  ci = [[] for _ in range(R)]
  for k in range(TOPK):
    for r in range(R):
      cv[r].append(tss[r][pl.ds(b_sc[r][k] * L, L)])
      ci[r].append(b_sc[r][k] * L + iota16)
  out = []
  ovs = [jnp.zeros((L,), jnp.float32) for _ in range(R)]
  ois = [jnp.zeros((L,), jnp.int32) for _ in range(R)]
  for k in range(TOPK):
    gv = [None] * R
    for r in range(R):
      m = cv[r][0]
      for t in range(1, TOPK):
        m = jnp.maximum(m, cv[r][t])
      gv[r] = jnp.max(m)
    for r in range(R):
      cidx = jnp.full((L,), N, jnp.int32)
      for t in range(TOPK):
        cidx = jnp.minimum(cidx, jnp.where(cv[r][t] == gv[r], ci[r][t], N))
      istar = jnp.min(cidx)
      ovs[r] = jnp.where(iota16 == k, gv[r], ovs[r])
      ois[r] = jnp.where(iota16 == k, istar, ois[r])
      for t in range(TOPK):
        cv[r][t] = jnp.where(ci[r][t] == istar, _NEG, cv[r][t])
  for r in range(R):
    out.append((ovs[r], ois[r]))
  return out


@functools.lru_cache(maxsize=1)
def _make_kernel():
  mesh = plsc.VectorSubcoreMesh(
      core_axis_name="c", subcore_axis_name="s",
      num_cores=NC, num_subcores=NS)

  @functools.partial(
      pl.kernel,
      out_type=[
          jax.ShapeDtypeStruct((ROWS, L), jnp.float32),
          jax.ShapeDtypeStruct((ROWS, L), jnp.int32),
      ],
      mesh=mesh,
      scratch_types=[
          pltpu.VMEM((N,), jnp.float32),
          pltpu.VMEM((N,), jnp.float32),
          pltpu.VMEM((L,), jnp.float32),
          pltpu.VMEM((L,), jnp.int32),
          pltpu.SemaphoreType.DMA,
          pltpu.SemaphoreType.DMA,
      ],
      compiler_params=pltpu.CompilerParams(needs_layout_passes=False),
  )
  def topk_kernel(logits_hbm, vals_hbm, idxs_hbm, ts_a, ts_b,
                  ov_ref, oi_ref, sem_a, sem_b):
    wid = lax.axis_index("s") * NC + lax.axis_index("c")
    row0 = wid * RPW
    iota16 = lax.iota(jnp.int32, L)

    def emit(ov, oi, row):
      ov_ref[...] = ov
      oi_ref[...] = oi
      pltpu.sync_copy(ov_ref, vals_hbm.at[row])
      pltpu.sync_copy(oi_ref, idxs_hbm.at[row])

    pltpu.make_async_copy(logits_hbm.at[row0], ts_a, sem_a).start()

    def pair_body(i, _):
      ra = row0 + 2 * i
      rb = ra + 1
      pltpu.make_async_copy(logits_hbm.at[ra], ts_a, sem_a).wait()
      pltpu.make_async_copy(logits_hbm.at[rb], ts_b, sem_b).start()
      sv_a = _pass1(ts_a, iota16)
      ((ov_a, oi_a),) = _rows_topk([ts_a], [sv_a], iota16)
      emit(ov_a, oi_a, ra)
      pltpu.make_async_copy(logits_hbm.at[rb], ts_b, sem_b).wait()

      @pl.when(i + 1 < RPW // 2)
      def _():
        pltpu.make_async_copy(logits_hbm.at[ra + 2], ts_a, sem_a).start()

      sv_b = _pass1(ts_b, iota16)
      ((ov_b, oi_b),) = _rows_topk([ts_b], [sv_b], iota16)
      emit(ov_b, oi_b, rb)
      return 0

    lax.fori_loop(0, RPW // 2, pair_body, 0)

  return topk_kernel


@jax.jit
def kernel(logits):
  vals, idxs = _make_kernel()(logits)
  return vals[:, :TOPK], idxs[:, :TOPK]


# speedup vs baseline: 1.1814x; 1.1814x over previous
import functools

import jax
import jax.numpy as jnp
from jax import lax
from jax.experimental import pallas as pl
from jax.experimental.pallas import tpu as pltpu
from jax.experimental.pallas import tpu_sc as plsc

ROWS = 128
N = 32768
TOPK = 10
NC = 2
NS = 16
L = 16
NW = NC * NS
RPW = ROWS // NW
SB = 256
NSB = N // SB
BPS = SB // L
NBLK = N // L

_NEG = float("-inf")


def _pass1(ts, iota16):
  sv = []
  for g in range(8):
    def sb_body(s_, ssm_g, g=g):
      sa = 2 * s_
      sb_i = sa + 1
      base_a = (g * 16 + sa) * SB
      base_b = (g * 16 + sb_i) * SB
      ma = ts[pl.ds(base_a, L)]
      mb = ts[pl.ds(base_b, L)]
      for j in range(1, SB // L):
        ma = jnp.maximum(ma, ts[pl.ds(base_a + j * L, L)])
        mb = jnp.maximum(mb, ts[pl.ds(base_b + j * L, L)])
      ssm_g = jnp.where(iota16 == sa, jnp.max(ma), ssm_g)
      ssm_g = jnp.where(iota16 == sb_i, jnp.max(mb), ssm_g)
      return ssm_g
    sv.append(lax.fori_loop(0, 8, sb_body, jnp.full((L,), _NEG, jnp.float32)))
  return sv


def _top10_cells(vecs, ids, nvec, sentinel, iota16):
  R = len(vecs)
  win = [jnp.zeros((L,), jnp.int32) for _ in range(R)]
  for k in range(TOPK):
    gv = [None] * R
    for r in range(R):
      m = vecs[r][0]
      for t in range(1, nvec):
        m = jnp.maximum(m, vecs[r][t])
      gv[r] = jnp.max(m)
    st = [None] * R
    for r in range(R):
      cidx = jnp.full((L,), sentinel, jnp.int32)
      for t in range(nvec):
        cidx = jnp.minimum(
            cidx, jnp.where(vecs[r][t] == gv[r], ids[r][t], sentinel))
      st[r] = jnp.min(cidx)
    for r in range(R):
      win[r] = jnp.where(iota16 == k, st[r], win[r])
      for t in range(nvec):
        vecs[r][t] = jnp.where(ids[r][t] == st[r], _NEG, vecs[r][t])
  return win


def _rows_topk(tss, svs, iota16):
  R = len(tss)

  sb_ids = [[iota16 + g * 16 for g in range(8)] for _ in range(R)]
  winsb = _top10_cells([list(sv) for sv in svs], sb_ids, 8, NSB, iota16)

  s_sc = [[jnp.max(jnp.where(iota16 == k, winsb[r], 0)) for k in range(TOPK)]
          for r in range(R)]

  bmv = [[] for _ in range(R)]
  bidv = [[] for _ in range(R)]
  for k in range(TOPK):
    for r in range(R):
      gidx = s_sc[r][k] * SB + iota16 * L
      bm = plsc.load_gather(tss[r], [gidx])
      for j in range(1, L):
        bm = jnp.maximum(bm, plsc.load_gather(tss[r], [gidx + j]))
      bmv[r].append(bm)
      bidv[r].append(s_sc[r][k] * BPS + iota16)

  winblk = _top10_cells(bmv, bidv, TOPK, NBLK, iota16)
  b_sc = [[jnp.max(jnp.where(iota16 == k, winblk[r], 0)) for k in range(TOPK)]
          for r in range(R)]

  cv = [[] for _ in range(R)]
  ci = [[] for _ in range(R)]
  for k in range(TOPK):
    for r in range(R):
      cv[r].append(tss[r][pl.ds(b_sc[r][k] * L, L)])
      ci[r].append(b_sc[r][k] * L + iota16)
  out = []
  ovs = [jnp.zeros((L,), jnp.float32) for _ in range(R)]
  ois = [jnp.zeros((L,), jnp.int32) for _ in range(R)]
  for k in range(TOPK):
    gv = [None] * R
    for r in range(R):
      m = cv[r][0]
      for t in range(1, TOPK):
        m = jnp.maximum(m, cv[r][t])
      gv[r] = jnp.max(m)
    for r in range(R):
      cidx = jnp.full((L,), N, jnp.int32)
      for t in range(TOPK):
        cidx = jnp.minimum(cidx, jnp.where(cv[r][t] == gv[r], ci[r][t], N))
      istar = jnp.min(cidx)
      ovs[r] = jnp.where(iota16 == k, gv[r], ovs[r])
      ois[r] = jnp.where(iota16 == k, istar, ois[r])
      for t in range(TOPK):
        cv[r][t] = jnp.where(ci[r][t] == istar, _NEG, cv[r][t])
  for r in range(R):
    out.append((ovs[r], ois[r]))
  return out


@functools.lru_cache(maxsize=1)
def _make_kernel():
  mesh = plsc.VectorSubcoreMesh(
      core_axis_name="c", subcore_axis_name="s",
      num_cores=NC, num_subcores=NS)

  @functools.partial(
      pl.kernel,
      out_type=[
          jax.ShapeDtypeStruct((ROWS, L), jnp.float32),
          jax.ShapeDtypeStruct((ROWS, L), jnp.int32),
      ],
      mesh=mesh,
      scratch_types=[
          pltpu.VMEM((N,), jnp.float32),
          pltpu.VMEM((N,), jnp.float32),
          pltpu.VMEM((L,), jnp.float32),
          pltpu.VMEM((L,), jnp.int32),
          pltpu.SemaphoreType.DMA,
          pltpu.SemaphoreType.DMA,
      ],
      compiler_params=pltpu.CompilerParams(needs_layout_passes=False),
  )
  def topk_kernel(logits_hbm, vals_hbm, idxs_hbm, ts_a, ts_b,
                  ov_ref, oi_ref, sem_a, sem_b):
    wid = lax.axis_index("s") * NC + lax.axis_index("c")
    row0 = wid * RPW
    iota16 = lax.iota(jnp.int32, L)

    def emit(ov, oi, row):
      ov_ref[...] = ov
      oi_ref[...] = oi
      pltpu.sync_copy(ov_ref, vals_hbm.at[row])
      pltpu.sync_copy(oi_ref, idxs_hbm.at[row])

    pltpu.make_async_copy(logits_hbm.at[row0], ts_a, sem_a).start()

    def pair_body(i, _):
      ra = row0 + 2 * i
      rb = ra + 1
      pltpu.make_async_copy(logits_hbm.at[ra], ts_a, sem_a).wait()
      pltpu.make_async_copy(logits_hbm.at[rb], ts_b, sem_b).start()
      sv_a = _pass1(ts_a, iota16)
      ((ov_a, oi_a),) = _rows_topk([ts_a], [sv_a], iota16)
      emit(ov_a, oi_a, ra)
      pltpu.make_async_copy(logits_hbm.at[rb], ts_b, sem_b).wait()

      @pl.when(i + 1 < RPW // 2)
      def _():
        pltpu.make_async_copy(logits_hbm.at[ra + 2], ts_a, sem_a).start()

      sv_b = _pass1(ts_b, iota16)
      ((ov_b, oi_b),) = _rows_topk([ts_b], [sv_b], iota16)
      emit(ov_b, oi_b, rb)
      return 0

    lax.fori_loop(0, RPW // 2, pair_body, 0)

  return topk_kernel


@jax.jit
def kernel(logits):
  vals, idxs = _make_kernel()(logits)
  return vals[:, :TOPK], idxs[:, :TOPK]
